# baseline (speedup 1.0000x reference)
import functools

import jax
import jax.numpy as jnp
from jax import lax
from jax.experimental import pallas as pl
from jax.experimental.pallas import tpu as pltpu
from jax.experimental.pallas import tpu_sc as plsc

_BATCH = 4096
_N_DENSE = 13
_N_SPARSE = 26
_VOCAB = 100000
_EMBED = 32

_NC = 2
_NS = 16
_NW = _NC * _NS
_BPW = _BATCH // _NW
_PACK = 128 // _EMBED
_ROWS128 = _N_SPARSE * _VOCAB // _PACK
_NSLOT = 6


def _sc_gather(table128, idx3):
    mesh = plsc.VectorSubcoreMesh(core_axis_name="c", subcore_axis_name="s")

    @functools.partial(
        pl.kernel,
        mesh=mesh,
        out_type=jax.ShapeDtypeStruct((_N_SPARSE, _BATCH, 128), jnp.float32),
        scratch_types=[
            pltpu.VMEM((_N_SPARSE, _BPW), jnp.int32),
            pltpu.VMEM((_NSLOT, _BPW, 128), jnp.float32),
            pltpu.SemaphoreType.DMA,
            pltpu.SemaphoreType.DMA,
        ],
    )
    def k(table_hbm, idx_hbm, out_hbm, idx_v, buf_v, gsem, osem):
        wid = lax.axis_index("s") * _NC + lax.axis_index("c")
        row0 = wid * _BPW
        pltpu.sync_copy(idx_hbm.at[wid], idx_v)
        gathers = [None] * _N_SPARSE
        for f in range(_NSLOT):
            gathers[f] = pltpu.async_copy(
                table_hbm.at[idx_v.at[f]], buf_v.at[f], gsem)
        for f in range(_NSLOT, _N_SPARSE):
            j = f - _NSLOT
            gathers[j].wait()
            oc = pltpu.async_copy(
                buf_v.at[j % _NSLOT],
                out_hbm.at[j].at[pl.ds(row0, _BPW)], osem)
            oc.wait()
            gathers[f] = pltpu.async_copy(
                table_hbm.at[idx_v.at[f]], buf_v.at[f % _NSLOT], gsem)
        tail = []
        for j in range(_N_SPARSE - _NSLOT, _N_SPARSE):
            gathers[j].wait()
            tail.append(pltpu.async_copy(
                buf_v.at[j % _NSLOT],
                out_hbm.at[j].at[pl.ds(row0, _BPW)], osem))
        for oc in tail:
            oc.wait()

    return k(table128, idx3)


_VQ = _VOCAB // _PACK


def _mlp_body(x_ref, sel_ref, inp_ref, w1_ref, b1_ref, w2_ref, b2_ref,
              w3_ref, b3_ref, w4_ref, wfull_ref, c0_ref, out_ref):
    f32 = jnp.float32
    h = jnp.zeros((x_ref.shape[1], 256), f32)
    for f in range(_N_SPARSE):
        z = lax.dot_general(x_ref[f], w1_ref[f], (((1,), (0,)), ((), ())),
                            preferred_element_type=f32)
        selc = sel_ref[:, f:f + 1]
        for s in range(_PACK):
            h = h + jnp.where(selc == s, z[:, 256 * s:256 * (s + 1)], 0.0)
    h = jnp.maximum(h + b1_ref[...], 0.0)
    h = lax.dot_general(h, w2_ref[...], (((1,), (0,)), ((), ())),
                        preferred_element_type=f32)
    h = jnp.maximum(h + b2_ref[...], 0.0)
    h = lax.dot_general(h, w3_ref[...], (((1,), (0,)), ((), ())),
                        preferred_element_type=f32)
    h = jnp.maximum(h + b3_ref[...], 0.0)
    deep = lax.dot_general(h, w4_ref[...], (((1,), (0,)), ((), ())),
                           preferred_element_type=f32)
    wide = lax.dot_general(inp_ref[...], wfull_ref[...], (((1,), (0,)), ((), ())),
                           preferred_element_type=f32)
    z = 0.5 * (deep + wide + c0_ref[0, 0])
    out_ref[...] = 1.0 / (1.0 + jnp.exp(-z))


def _tc_mlp(x3, sel, inputs, w1cat, b1, w2, b2, w3, b3, w4, wfull, c0):
    bb = 512
    nb = _BATCH // bb
    d_in = inputs.shape[1]
    return pl.pallas_call(
        _mlp_body,
        grid=(nb,),
        in_specs=[
            pl.BlockSpec((_N_SPARSE, bb, 128), lambda i: (0, i, 0)),
            pl.BlockSpec((bb, _N_SPARSE), lambda i: (i, 0)),
            pl.BlockSpec((bb, d_in), lambda i: (i, 0)),
            pl.BlockSpec((_N_SPARSE, 128, 4 * 256), lambda i: (0, 0, 0)),
            pl.BlockSpec((1, 256), lambda i: (0, 0)),
            pl.BlockSpec((256, 128), lambda i: (0, 0)),
            pl.BlockSpec((1, 128), lambda i: (0, 0)),
            pl.BlockSpec((128, 64), lambda i: (0, 0)),
            pl.BlockSpec((1, 64), lambda i: (0, 0)),
            pl.BlockSpec((64, 1), lambda i: (0, 0)),
            pl.BlockSpec((d_in, 1), lambda i: (0, 0)),
            pl.BlockSpec((1, 1), lambda i: (0, 0)),
        ],
        out_specs=pl.BlockSpec((bb, 1), lambda i: (i, 0)),
        out_shape=jax.ShapeDtypeStruct((_BATCH, 1), jnp.float32),
    )(x3, sel, inputs, w1cat, b1, w2, b2, w3, b3, w4, wfull, c0)


def kernel(inputs, tables, w_wide, b_wide, deep_Ws, deep_Bs):
    sparse_idx = inputs[:, _N_DENSE:_N_DENSE + _N_SPARSE].astype(jnp.int32)
    sel = sparse_idx // _VQ
    g2 = sparse_idx % _VQ + (
        jnp.arange(_N_SPARSE, dtype=jnp.int32) * _VQ)[None, :]
    idx3 = g2.T.reshape(_N_SPARSE, _NW, _BPW).transpose(1, 0, 2)
    tt6 = jnp.transpose(tables, (0, 2, 1)).reshape(_N_SPARSE, 128, _VQ)
    table128 = jnp.swapaxes(tt6, 1, 2).reshape(_ROWS128, 128)

    w1r = deep_Ws[0].reshape(_N_SPARSE, _EMBED, 256)
    w1cat = jnp.zeros((_N_SPARSE, 128, 4 * 256), jnp.float32)
    for s in range(_PACK):
        w1cat = w1cat.at[:, s::_PACK, 256 * s:256 * (s + 1)].set(w1r)

    wfull = jnp.concatenate(
        [w_wide[:_N_DENSE],
         jnp.zeros((_N_SPARSE, 1), jnp.float32),
         w_wide[_N_DENSE:]], axis=0)
    w4 = deep_Ws[3]
    c0 = (b_wide[0] + deep_Bs[3][0]).reshape(1, 1)
    b1 = deep_Bs[0].reshape(1, -1)
    b2 = deep_Bs[1].reshape(1, -1)
    b3 = deep_Bs[2].reshape(1, -1)

    x3 = _sc_gather(table128, idx3)

    return _tc_mlp(x3, sel, inputs, w1cat, b1, deep_Ws[1], b2,
                   deep_Ws[2], b3, w4, wfull, c0)


# speedup vs baseline: 1.2710x; 1.2710x over previous
import functools

import jax
import jax.numpy as jnp
from jax import lax
from jax.experimental import pallas as pl
from jax.experimental.pallas import tpu as pltpu
from jax.experimental.pallas import tpu_sc as plsc

_BATCH = 4096
_N_DENSE = 13
_N_SPARSE = 26
_VOCAB = 100000
_EMBED = 32

_NC = 2
_NS = 16
_NW = _NC * _NS
_BPW = _BATCH // _NW
_PACK = 128 // _EMBED
_ROWS128 = _N_SPARSE * _VOCAB // _PACK
_NSLOT = 6


def _sc_gather(table128, idx3):
    mesh = plsc.VectorSubcoreMesh(core_axis_name="c", subcore_axis_name="s")

    @functools.partial(
        pl.kernel,
        mesh=mesh,
        out_type=jax.ShapeDtypeStruct((_N_SPARSE, _BATCH, 128), jnp.float32),
        scratch_types=[
            pltpu.VMEM((_N_SPARSE, _BPW), jnp.int32),
            pltpu.VMEM((_NSLOT, _BPW, 128), jnp.float32),
            pltpu.SemaphoreType.DMA,
            pltpu.SemaphoreType.DMA,
        ],
    )
    def k(table_hbm, idx_hbm, out_hbm, idx_v, buf_v, gsem, osem):
        wid = lax.axis_index("s") * _NC + lax.axis_index("c")
        row0 = wid * _BPW
        pltpu.sync_copy(idx_hbm.at[wid], idx_v)
        gathers = [None] * _N_SPARSE
        for f in range(_NSLOT):
            gathers[f] = pltpu.async_copy(
                table_hbm.at[idx_v.at[f]], buf_v.at[f], gsem)
        for f in range(_NSLOT, _N_SPARSE):
            j = f - _NSLOT
            gathers[j].wait()
            oc = pltpu.async_copy(
                buf_v.at[j % _NSLOT],
                out_hbm.at[j].at[pl.ds(row0, _BPW)], osem)
            oc.wait()
            gathers[f] = pltpu.async_copy(
                table_hbm.at[idx_v.at[f]], buf_v.at[f % _NSLOT], gsem)
        tail = []
        for j in range(_N_SPARSE - _NSLOT, _N_SPARSE):
            gathers[j].wait()
            tail.append(pltpu.async_copy(
                buf_v.at[j % _NSLOT],
                out_hbm.at[j].at[pl.ds(row0, _BPW)], osem))
        for oc in tail:
            oc.wait()

    return k(table128, idx3)


_VQ = _VOCAB // _PACK


def _mlp_body(x_ref, inp_ref, w1_ref, b1_ref, w2_ref, b2_ref,
              w3_ref, b3_ref, w4_ref, wfull_ref, c0_ref, out_ref):
    f32 = jnp.float32
    h = jnp.zeros((x_ref.shape[1], 256), f32)
    for f in range(_N_SPARSE):
        h = h + lax.dot_general(x_ref[f], w1_ref[f], (((1,), (0,)), ((), ())),
                                preferred_element_type=f32)
    h = jnp.maximum(h + b1_ref[...], 0.0)
    h = lax.dot_general(h, w2_ref[...], (((1,), (0,)), ((), ())),
                        preferred_element_type=f32)
    h = jnp.maximum(h + b2_ref[...], 0.0)
    h = lax.dot_general(h, w3_ref[...], (((1,), (0,)), ((), ())),
                        preferred_element_type=f32)
    h = jnp.maximum(h + b3_ref[...], 0.0)
    deep = lax.dot_general(h, w4_ref[...], (((1,), (0,)), ((), ())),
                           preferred_element_type=f32)
    wide = lax.dot_general(inp_ref[...], wfull_ref[...], (((1,), (0,)), ((), ())),
                           preferred_element_type=f32)
    z = 0.5 * (deep + wide + c0_ref[0, 0])
    out_ref[...] = 1.0 / (1.0 + jnp.exp(-z))


def _tc_mlp(x3, inputs, w1cat, b1, w2, b2, w3, b3, w4, wfull, c0):
    bb = 512
    nb = _BATCH // bb
    d_in = inputs.shape[1]
    return pl.pallas_call(
        _mlp_body,
        grid=(nb,),
        in_specs=[
            pl.BlockSpec((_N_SPARSE, bb, 128), lambda i: (0, i, 0)),
            pl.BlockSpec((bb, d_in), lambda i: (i, 0)),
            pl.BlockSpec((_N_SPARSE, 128, 256), lambda i: (0, 0, 0)),
            pl.BlockSpec((1, 256), lambda i: (0, 0)),
            pl.BlockSpec((256, 128), lambda i: (0, 0)),
            pl.BlockSpec((1, 128), lambda i: (0, 0)),
            pl.BlockSpec((128, 64), lambda i: (0, 0)),
            pl.BlockSpec((1, 64), lambda i: (0, 0)),
            pl.BlockSpec((64, 1), lambda i: (0, 0)),
            pl.BlockSpec((d_in, 1), lambda i: (0, 0)),
            pl.BlockSpec((1, 1), lambda i: (0, 0)),
        ],
        out_specs=pl.BlockSpec((bb, 1), lambda i: (i, 0)),
        out_shape=jax.ShapeDtypeStruct((_BATCH, 1), jnp.float32),
    )(x3, inputs, w1cat, b1, w2, b2, w3, b3, w4, wfull, c0)


def kernel(inputs, tables, w_wide, b_wide, deep_Ws, deep_Bs):
    sparse_idx = inputs[:, _N_DENSE:_N_DENSE + _N_SPARSE].astype(jnp.int32)
    g2 = sparse_idx + (
        jnp.arange(_N_SPARSE, dtype=jnp.int32) * _VOCAB)[None, :]
    idx3 = g2.T.reshape(_N_SPARSE, _NW, _BPW).transpose(1, 0, 2)
    table128 = jnp.pad(tables.reshape(_N_SPARSE * _VOCAB, _EMBED),
                       ((0, 0), (0, 128 - _EMBED)))

    w1r = deep_Ws[0].reshape(_N_SPARSE, _EMBED, 256)
    w1cat = jnp.pad(w1r, ((0, 0), (0, 128 - _EMBED), (0, 0)))

    wfull = jnp.concatenate(
        [w_wide[:_N_DENSE],
         jnp.zeros((_N_SPARSE, 1), jnp.float32),
         w_wide[_N_DENSE:]], axis=0)
    w4 = deep_Ws[3]
    c0 = (b_wide[0] + deep_Bs[3][0]).reshape(1, 1)
    b1 = deep_Bs[0].reshape(1, -1)
    b2 = deep_Bs[1].reshape(1, -1)
    b3 = deep_Bs[2].reshape(1, -1)

    x3 = _sc_gather(table128, idx3)

    return _tc_mlp(x3, inputs, w1cat, b1, deep_Ws[1], b2,
                   deep_Ws[2], b3, w4, wfull, c0)
